# baseline (speedup 1.0000x reference)
import jax
import jax.numpy as jnp
from jax.experimental import pallas as pl
from jax.experimental.pallas import tpu as pltpu

_NE = 1024
_D = 64
_P = 4096


def _body(x_ref, e_ref, o_ref, e2_ref):
    e = e_ref[...]

    @pl.when((pl.program_id(0) == 0) & (pl.program_id(1) == 0))
    def _init():
        e2_ref[...] = jnp.sum(e * e, axis=1, keepdims=True)

    e2 = e2_ref[...]
    e2x = e + e
    _H = _P // 2
    jidx = jax.lax.broadcasted_iota(jnp.int32, (_NE, _H), 0).astype(jnp.float32)
    for k in range(2):
        x = x_ref[0, :, pl.ds(k * _H, _H)]
        x2 = jnp.sum(x * x, axis=0, keepdims=True)
        mm2 = jax.lax.dot_general(e2x, x, (((1,), (0,)), ((), ())))
        dist = (x2 + e2) - mm2
        m = jnp.min(dist, axis=0, keepdims=True)
        idx = jnp.min(jnp.where(dist == m, jidx, float(_NE)), axis=0,
                      keepdims=True)
        oh = (jidx == idx).astype(jnp.float32)
        q = jax.lax.dot_general(e, oh, (((0,), (0,)), ((), ())))
        o_ref[0, :, pl.ds(k * _H, _H)] = x + (q - x)


def kernel(inputs, embedding):
    b, c, h, w = inputs.shape
    xf = inputs.reshape(b, c, h * w)
    npix = h * w
    out = pl.pallas_call(
        _body,
        grid=(b, npix // _P),
        in_specs=[pl.BlockSpec((1, c, _P), lambda i, j: (i, 0, j)),
                  pl.BlockSpec((_NE, _D), lambda i, j: (0, 0))],
        out_specs=pl.BlockSpec((1, c, _P), lambda i, j: (i, 0, j)),
        out_shape=jax.ShapeDtypeStruct((b, c, npix), jnp.float32),
        scratch_shapes=[pltpu.VMEM((_NE, 1), jnp.float32)],
    )(xf, embedding)
    return out.reshape(b, c, h, w)


# speedup vs baseline: 1.0195x; 1.0195x over previous
import jax
import jax.numpy as jnp
from jax.experimental import pallas as pl
from jax.experimental.pallas import tpu as pltpu

_NE = 1024
_D = 64
_P = 4096


def _body(x_ref, e_ref, o_ref, e2_ref):
    e = e_ref[...]

    @pl.when((pl.program_id(0) == 0) & (pl.program_id(1) == 0))
    def _init():
        e2_ref[...] = jnp.sum(e * e, axis=1, keepdims=True)

    x = x_ref[0]
    x2 = jnp.sum(x * x, axis=0, keepdims=True)
    e2 = e2_ref[...]
    mm2 = jax.lax.dot_general(e + e, x, (((1,), (0,)), ((), ())))
    dist = (x2 + e2) - mm2
    m = jnp.min(dist, axis=0, keepdims=True)
    jidx = jax.lax.broadcasted_iota(jnp.int32, (_NE, _P), 0).astype(jnp.float32)
    idx = jnp.min(jnp.where(dist == m, jidx, float(_NE)), axis=0,
                  keepdims=True)
    oh = (jidx == idx).astype(jnp.float32)
    q = jax.lax.dot_general(e, oh, (((0,), (0,)), ((), ())))
    o_ref[0] = x + (q - x)


def kernel(inputs, embedding):
    b, c, h, w = inputs.shape
    xf = inputs.reshape(b, c, h * w)
    npix = h * w
    out = pl.pallas_call(
        _body,
        grid=(b, npix // _P),
        in_specs=[pl.BlockSpec((1, c, _P), lambda i, j: (i, 0, j)),
                  pl.BlockSpec((_NE, _D), lambda i, j: (0, 0))],
        out_specs=pl.BlockSpec((1, c, _P), lambda i, j: (i, 0, j)),
        out_shape=jax.ShapeDtypeStruct((b, c, npix), jnp.float32),
        scratch_shapes=[pltpu.VMEM((_NE, 1), jnp.float32)],
    )(xf, embedding)
    return out.reshape(b, c, h, w)


# speedup vs baseline: 1.1540x; 1.1319x over previous
import jax
import jax.numpy as jnp
from jax.experimental import pallas as pl
from jax.experimental.pallas import tpu as pltpu

_NE = 1024
_D = 64
_P = 4096


def _body(x_ref, e_ref, o_ref, e2_ref):
    e = e_ref[...]

    @pl.when((pl.program_id(0) == 0) & (pl.program_id(1) == 0))
    def _init():
        e2_ref[...] = jnp.sum(e * e, axis=1, keepdims=True)

    x = x_ref[0]
    x2 = jnp.sum(x * x, axis=0, keepdims=True)
    e2 = e2_ref[...]
    mm2 = jax.lax.dot_general(e + e, x, (((1,), (0,)), ((), ())))
    dist = (x2 + e2) - mm2
    idxi = jnp.argmin(dist, axis=0)[None, :]
    jidxi = jax.lax.broadcasted_iota(jnp.int32, (_NE, _P), 0)
    oh = (jidxi == idxi).astype(jnp.float32)
    q = jax.lax.dot_general(e, oh, (((0,), (0,)), ((), ())))
    o_ref[0] = x + (q - x)


def kernel(inputs, embedding):
    b, c, h, w = inputs.shape
    xf = inputs.reshape(b, c, h * w)
    npix = h * w
    out = pl.pallas_call(
        _body,
        grid=(b, npix // _P),
        in_specs=[pl.BlockSpec((1, c, _P), lambda i, j: (i, 0, j)),
                  pl.BlockSpec((_NE, _D), lambda i, j: (0, 0))],
        out_specs=pl.BlockSpec((1, c, _P), lambda i, j: (i, 0, j)),
        out_shape=jax.ShapeDtypeStruct((b, c, npix), jnp.float32),
        scratch_shapes=[pltpu.VMEM((_NE, 1), jnp.float32)],
    )(xf, embedding)
    return out.reshape(b, c, h, w)
